# Initial kernel scaffold; baseline (speedup 1.0000x reference)
import jax
import jax.numpy as jnp
from jax.experimental import pallas as pl


def kernel(inputs, embedding):
    raise NotImplementedError("write your pallas kernel here")


# speedup vs baseline: 2.0605x; 2.0605x over previous
import jax
import jax.numpy as jnp
from jax.experimental import pallas as pl
from jax.experimental.pallas import tpu as pltpu
from jax.experimental.pallas import tpu_sc as plsc

VOCAB = 1024
DIM = 64
N_ROWS = 32 * 576
BLOCK = 2304
NB = N_ROWS // BLOCK
GATHER_WINDOW = 128


def _tc_body(x_ref, et_ref, a2_ref, b2_ref,
             idx_ref, loss_ref, perp_ref,
             counts_ref, se_ref):
    step = pl.program_id(0)

    x = x_ref[...]
    xe = jnp.dot(x, et_ref[...], preferred_element_type=jnp.float32)
    a2 = a2_ref[...]
    b2 = b2_ref[...]
    d = (a2 + b2) - 4.0 * xe

    mv = jnp.min(d, axis=1, keepdims=True)
    lane = jax.lax.broadcasted_iota(jnp.int32, d.shape, 1)
    idx = jnp.min(jnp.where(d == mv, lane, jnp.int32(2 ** 30)),
                  axis=1, keepdims=True)
    idx_ref[...] = idx

    @pl.when(step == 0)
    def _():
        counts_ref[...] = jnp.zeros_like(counts_ref)
        se_ref[0] = 0.0

    onehot = lane == idx
    counts_ref[...] += jnp.sum(onehot.astype(jnp.float32), axis=0,
                               keepdims=True)
    se_ref[0] += (jnp.sum(a2)
                  + jnp.sum(jnp.where(onehot, b2 - 2.0 * xe, 0.0)))

    @pl.when(step == NB - 1)
    def _():
        avg = counts_ref[...] / jnp.float32(N_ROWS)
        ent = jnp.sum(avg * jnp.log(avg + 1e-10), axis=1, keepdims=True)
        perp_ref[...] = jnp.exp(-ent)
        loss_ref[...] = jnp.reshape(
            1.25 * se_ref[0] / jnp.float32(N_ROWS * DIM), (1, 1))


def _tc_quantize(x, et, a2, b2):
    return pl.pallas_call(
        _tc_body,
        grid=(NB,),
        in_specs=[
            pl.BlockSpec((BLOCK, DIM), lambda i: (i, 0)),
            pl.BlockSpec((DIM, VOCAB), lambda i: (0, 0)),
            pl.BlockSpec((BLOCK, 1), lambda i: (i, 0)),
            pl.BlockSpec((1, VOCAB), lambda i: (0, 0)),
        ],
        out_specs=[
            pl.BlockSpec((BLOCK, 1), lambda i: (i, 0)),
            pl.BlockSpec((1, 1), lambda i: (0, 0)),
            pl.BlockSpec((1, 1), lambda i: (0, 0)),
        ],
        out_shape=[
            jax.ShapeDtypeStruct((N_ROWS, 1), jnp.int32),
            jax.ShapeDtypeStruct((1, 1), jnp.float32),
            jax.ShapeDtypeStruct((1, 1), jnp.float32),
        ],
        scratch_shapes=[
            pltpu.VMEM((1, VOCAB), jnp.float32),
            pltpu.SMEM((1,), jnp.float32),
        ],
    )(x, et, a2, b2)


def _sc_gather(emb_padded, idx_flat):
    mesh = plsc.VectorSubcoreMesh(core_axis_name="core",
                                  subcore_axis_name="subcore")

    @pl.kernel(out_type=jax.ShapeDtypeStruct((N_ROWS, 128), jnp.float32),
               mesh=mesh)
    def k(emb_hbm, i_hbm, o_hbm):
        def body(i_vmem, o_vmem):
            pltpu.sync_copy(emb_hbm.at[i_vmem.at[0]], o_vmem)

        pltpu.emit_pipeline(
            body,
            grid=(N_ROWS // GATHER_WINDOW,),
            in_specs=[pl.BlockSpec((1, GATHER_WINDOW),
                                   index_map=lambda i: (0, i))],
            out_specs=[pl.BlockSpec((GATHER_WINDOW, 128),
                                    index_map=lambda i: (i, 0))],
            core_axis_name="subcore",
            dimension_semantics=(pltpu.PARALLEL,),
        )(i_hbm, o_hbm)

    return k(emb_padded, idx_flat)


def kernel(inputs, embedding):
    input_shape = inputs.shape
    x = inputs.reshape(-1, DIM)
    a2 = jnp.sum(x ** 2, axis=1, keepdims=True)
    b2 = jnp.sum(embedding ** 2, axis=1)

    idx2, loss, perp = _tc_quantize(x, embedding.T, a2,
                                    b2.reshape(1, VOCAB))
    idx = idx2.reshape(-1)
    emb_padded = jnp.pad(embedding, ((0, 0), (0, 128 - DIM)))
    quantized = _sc_gather(emb_padded, idx.reshape(1, N_ROWS))[:, :DIM]

    quantized_st = quantized.reshape(input_shape)
    enc_idx_out = idx.reshape(input_shape[0], input_shape[1])
    return (quantized_st, enc_idx_out, loss.reshape(()), perp.reshape(()))
